# baseline (speedup 1.0000x reference)
import functools

import jax
import jax.numpy as jnp
from jax.experimental import pallas as pl
from jax.experimental.pallas import tpu as pltpu

_EPS = 1e-07


def _roll(x, k):
    return pltpu.roll(x, x.shape[1] - k, 1)


def _group_clsmax(x):
    t2 = jnp.maximum(x, _roll(x, 1))
    t4 = jnp.maximum(t2, _roll(t2, 2))
    t8 = jnp.maximum(t4, _roll(t4, 4))
    t16 = jnp.maximum(t8, _roll(t8, 8))
    return jnp.maximum(_roll(t16, 1), _roll(t4, 17))


def _group_sum21(x):
    s2 = x + _roll(x, 1)
    s4 = s2 + _roll(s2, 2)
    s8 = s4 + _roll(s4, 4)
    s16 = s8 + _roll(s8, 8)
    s20 = s16 + _roll(s4, 16)
    return s20 + _roll(x, 20)


def _isd_kernel(xa_ref, xb_ref, qa_ref, qb_ref, loss_ref, acc_ref, *, ngrid, cls):
    i = pl.program_id(0)

    @pl.when(i == 0)
    def _init():
        acc_ref[0] = 0.0
        acc_ref[1] = 0.0

    xa = xa_ref[0]
    xb = xb_ref[0]

    lane = jax.lax.broadcasted_iota(jnp.int32, xa.shape, 1)
    gstart = (lane % cls) == 0

    ma = _group_clsmax(xa) > xa
    mb = _group_clsmax(xb) > xb

    def side(x, q_ref, only):
        t = x + _EPS
        ke = t * (jnp.log(t) - jnp.log(q_ref[0] + _EPS))
        w = jnp.logical_and(only, gstart).astype(jnp.float32)
        return _group_sum21(ke) * w, w

    sa, wa = side(xa, qa_ref, jnp.logical_and(ma, jnp.logical_not(mb)))
    sb, wb = side(xb, qb_ref, jnp.logical_and(mb, jnp.logical_not(ma)))
    acc_ref[0] += jnp.sum(sa + sb)
    acc_ref[1] += jnp.sum(wa + wb)

    @pl.when(i == ngrid - 1)
    def _finalize():
        total = acc_ref[0]
        cnt = acc_ref[1]
        val = jnp.where(cnt > 0.0, total / jnp.maximum(cnt, 1.0), 0.0)
        loss_ref[...] = jnp.full((1, 1), val, dtype=jnp.float32)


def kernel(args, lam, conf, loc, conf_mix, loc_mix):
    B, P, C = conf.shape
    L = C * 64
    rows = (B * P * C) // L
    half_rows = rows // 2
    RB = 59
    ngrid = half_rows // RB

    X = conf.reshape(rows // RB, RB, L)
    Q = conf_mix.reshape(rows // RB, RB, L)

    blk = (1, RB, L)
    spec_a = pl.BlockSpec(blk, lambda i: (i, 0, 0))
    spec_b = pl.BlockSpec(blk, lambda i, hb=half_rows // RB: (i + hb, 0, 0))

    loss = pl.pallas_call(
        functools.partial(_isd_kernel, ngrid=ngrid, cls=C),
        grid=(ngrid,),
        in_specs=[spec_a, spec_b, spec_a, spec_b],
        out_specs=pl.BlockSpec((1, 1), lambda i: (0, 0)),
        out_shape=jax.ShapeDtypeStruct((1, 1), jnp.float32),
        scratch_shapes=[pltpu.SMEM((2,), jnp.float32)],
    )(X, X, Q, Q)

    return (jnp.zeros((1,), dtype=jnp.float32), loss[0, 0])


# speedup vs baseline: 23.6790x; 23.6790x over previous
import functools

import jax
import jax.numpy as jnp
from jax.experimental import pallas as pl
from jax.experimental.pallas import tpu as pltpu

_EPS = 1e-07


def _isd_kernel(x_ref, q_ref, loss_ref, acc_ref, *, ngrid, pb, p_total):
    j = pl.program_id(0)

    @pl.when(j == 0)
    def _init():
        acc_ref[0] = 0.0
        acc_ref[1] = 0.0

    x = x_ref[...]
    q = q_ref[...]
    C, B, _ = x.shape

    bg = x[0]
    clsmax = x[1]
    for c in range(2, C):
        clsmax = jnp.maximum(clsmax, x[c])
    leftf = (clsmax > bg).astype(jnp.float32)
    rightf = pltpu.roll(leftf, B // 2, 0)

    lane = jax.lax.broadcasted_iota(jnp.int32, bg.shape, 1)
    valid = (lane + j * pb) < p_total
    w = jnp.logical_and((leftf - rightf) > 0.5, valid)

    ks = None
    for c in range(C):
        t = x[c] + _EPS
        term = t * jnp.log(t / (q[c] + _EPS))
        ks = term if ks is None else ks + term

    acc_ref[0] += jnp.sum(jnp.where(w, ks, 0.0))
    acc_ref[1] += jnp.sum(jnp.where(w, 1.0, 0.0))

    @pl.when(j == ngrid - 1)
    def _finalize():
        total = acc_ref[0]
        cnt = acc_ref[1]
        val = jnp.where(cnt > 0.0, total / jnp.maximum(cnt, 1.0), 0.0)
        loss_ref[...] = jnp.full((1, 1), val, dtype=jnp.float32)


def kernel(args, lam, conf, loc, conf_mix, loc_mix):
    B, P, C = conf.shape
    PB = 512
    ngrid = pl.cdiv(P, PB)

    X = jnp.transpose(conf, (2, 0, 1))
    Q = jnp.transpose(conf_mix, (2, 0, 1))

    blk = (C, B, PB)
    spec = pl.BlockSpec(blk, lambda j: (0, 0, j))

    loss = pl.pallas_call(
        functools.partial(_isd_kernel, ngrid=ngrid, pb=PB, p_total=P),
        grid=(ngrid,),
        in_specs=[spec, spec],
        out_specs=pl.BlockSpec((1, 1), lambda j: (0, 0)),
        out_shape=jax.ShapeDtypeStruct((1, 1), jnp.float32),
        scratch_shapes=[pltpu.SMEM((2,), jnp.float32)],
    )(X, Q)

    return (jnp.zeros((1,), dtype=jnp.float32), loss[0, 0])


# speedup vs baseline: 28.4947x; 1.2034x over previous
import functools

import jax
import jax.numpy as jnp
from jax.experimental import pallas as pl
from jax.experimental.pallas import tpu as pltpu

_EPS = 1e-07


def _isd_kernel(x_ref, q_ref, loss_ref, acc_ref, *, ngrid, pb, p_total):
    j = pl.program_id(0)

    @pl.when(j == 0)
    def _init():
        acc_ref[0] = 0.0
        acc_ref[1] = 0.0

    x = x_ref[...]
    q = q_ref[...]
    C, B, _ = x.shape

    bg = x[0]
    clsmax = x[1]
    for c in range(2, C):
        clsmax = jnp.maximum(clsmax, x[c])
    leftf = (clsmax > bg).astype(jnp.float32)
    rightf = pltpu.roll(leftf, B // 2, 0)

    lane = jax.lax.broadcasted_iota(jnp.int32, bg.shape, 1)
    valid = (lane + j * pb) < p_total
    w = jnp.logical_and((leftf - rightf) > 0.5, valid)

    ks = None
    for c in range(C):
        t = x[c] + _EPS
        term = t * jnp.log(t / (q[c] + _EPS))
        ks = term if ks is None else ks + term

    acc_ref[0] += jnp.sum(jnp.where(w, ks, 0.0))
    acc_ref[1] += jnp.sum(jnp.where(w, 1.0, 0.0))

    @pl.when(j == ngrid - 1)
    def _finalize():
        total = acc_ref[0]
        cnt = acc_ref[1]
        val = jnp.where(cnt > 0.0, total / jnp.maximum(cnt, 1.0), 0.0)
        loss_ref[...] = jnp.full((1, 1), val, dtype=jnp.float32)


def kernel(args, lam, conf, loc, conf_mix, loc_mix):
    B, P, C = conf.shape
    PB = 1024
    ngrid = pl.cdiv(P, PB)

    X = jnp.transpose(conf, (2, 0, 1))
    Q = jnp.transpose(conf_mix, (2, 0, 1))

    blk = (C, B, PB)
    spec = pl.BlockSpec(blk, lambda j: (0, 0, j))

    loss = pl.pallas_call(
        functools.partial(_isd_kernel, ngrid=ngrid, pb=PB, p_total=P),
        grid=(ngrid,),
        in_specs=[spec, spec],
        out_specs=pl.BlockSpec((1, 1), lambda j: (0, 0)),
        out_shape=jax.ShapeDtypeStruct((1, 1), jnp.float32),
        scratch_shapes=[pltpu.SMEM((2,), jnp.float32)],
    )(X, Q)

    return (jnp.zeros((1,), dtype=jnp.float32), loss[0, 0])
